# Initial kernel scaffold; baseline (speedup 1.0000x reference)
import jax
import jax.numpy as jnp
from jax.experimental import pallas as pl


def kernel(feat, adj, weight, bias):
    raise NotImplementedError("write your pallas kernel here")


# speedup vs baseline: 1.0347x; 1.0347x over previous
import functools

import jax
import jax.numpy as jnp
from jax.experimental import pallas as pl
from jax.experimental.pallas import tpu as pltpu

N = 10000
D_IN = 128
D_OUT = 128
BR = 200


def _gcn_kernel(feat_ref, adj_ref, weight_ref, bias_ref, out_ref, support_ref):
    r = pl.program_id(0)

    @pl.when(r == 0)
    def _():
        support_ref[...] = jnp.dot(
            feat_ref[...], weight_ref[...], preferred_element_type=jnp.float32
        )

    out_ref[...] = (
        jnp.dot(adj_ref[...], support_ref[...], preferred_element_type=jnp.float32)
        + bias_ref[...]
    )


@jax.jit
def kernel(feat, adj, weight, bias):
    bias2d = bias.reshape(1, D_OUT)
    grid = (N // BR,)
    out = pl.pallas_call(
        _gcn_kernel,
        grid=grid,
        in_specs=[
            pl.BlockSpec((N, D_IN), lambda r: (0, 0)),
            pl.BlockSpec((BR, N), lambda r: (r, 0)),
            pl.BlockSpec((D_IN, D_OUT), lambda r: (0, 0)),
            pl.BlockSpec((1, D_OUT), lambda r: (0, 0)),
        ],
        out_specs=pl.BlockSpec((BR, D_OUT), lambda r: (r, 0)),
        out_shape=jax.ShapeDtypeStruct((N, D_OUT), jnp.float32),
        scratch_shapes=[pltpu.VMEM((N, D_OUT), jnp.float32)],
    )(feat, adj, weight, bias2d)
    return out


# speedup vs baseline: 1.0360x; 1.0013x over previous
import functools

import jax
import jax.numpy as jnp
from jax.experimental import pallas as pl
from jax.experimental.pallas import tpu as pltpu

N = 10000
D_IN = 128
D_OUT = 128
BR = 400


def _gcn_kernel(feat_ref, adj_ref, weight_ref, bias_ref, out_ref, support_ref):
    r = pl.program_id(0)

    @pl.when(r == 0)
    def _():
        support_ref[...] = jnp.dot(
            feat_ref[...], weight_ref[...], preferred_element_type=jnp.float32
        )

    out_ref[...] = (
        jnp.dot(adj_ref[...], support_ref[...], preferred_element_type=jnp.float32)
        + bias_ref[...]
    )


@jax.jit
def kernel(feat, adj, weight, bias):
    bias2d = bias.reshape(1, D_OUT)
    grid = (N // BR,)
    out = pl.pallas_call(
        _gcn_kernel,
        grid=grid,
        in_specs=[
            pl.BlockSpec((N, D_IN), lambda r: (0, 0)),
            pl.BlockSpec((BR, N), lambda r: (r, 0)),
            pl.BlockSpec((D_IN, D_OUT), lambda r: (0, 0)),
            pl.BlockSpec((1, D_OUT), lambda r: (0, 0)),
        ],
        out_specs=pl.BlockSpec((BR, D_OUT), lambda r: (r, 0)),
        out_shape=jax.ShapeDtypeStruct((N, D_OUT), jnp.float32),
        scratch_shapes=[pltpu.VMEM((N, D_OUT), jnp.float32)],
    )(feat, adj, weight, bias2d)
    return out
